# baseline (speedup 1.0000x reference)
import functools

import jax
import jax.numpy as jnp
from jax import lax
from jax.experimental import pallas as pl
from jax.experimental.pallas import tpu as pltpu
from jax.experimental.pallas import tpu_sc as plsc

D = 64
IMG_DIM = 1536
BATCH = 16384
NROWS = 1000000

_info = plsc.get_sparse_core_info()
_NC = _info.num_cores
_NS = _info.num_subcores
_NW = _NC * _NS

_SLABW = 512
_NSLAB = (NROWS + _SLABW - 1) // _SLABW
_ALIGNED_LAST = ((NROWS - _SLABW) // 128) * 128
_TAIL0 = (NROWS // 128) * 128
_NTAIL = NROWS - _TAIL0
_TRASH = BATCH
_CAP = 49152
_PERC = _CAP // _NC

_sc_mesh = plsc.VectorSubcoreMesh(core_axis_name="c", subcore_axis_name="s")


def _iota16():
    return lax.iota(jnp.int32, 16)


def _splat(x):
    return jnp.full((16,), 1, jnp.int32) * x


@functools.partial(
    pl.kernel,
    mesh=_sc_mesh,
    out_type=(
        jax.ShapeDtypeStruct((_CAP * D,), jnp.float32),
        jax.ShapeDtypeStruct((_CAP,), jnp.int32),
    ),
    scratch_types=[
        pltpu.VMEM((BATCH,), jnp.int32),
        pltpu.VMEM((BATCH,), jnp.int32),
        pltpu.VMEM((BATCH,), jnp.int32),
        pltpu.VMEM((2, 64, _SLABW), jnp.float32),
        pltpu.VMEM((48,), jnp.int32),
        pltpu.VMEM((48,), jnp.int32),
        pltpu.VMEM((2 * 16 * D,), jnp.float32),
        pltpu.VMEM((2 * 16,), jnp.int32),
        pltpu.VMEM((1536,), jnp.int32),
        pltpu.VMEM((D, _NTAIL), jnp.float32),
        pltpu.SMEM((8,), jnp.int32),
        pltpu.SemaphoreType.DMA,
        pltpu.SemaphoreType.DMA,
    ],
    compiler_params=pltpu.CompilerParams(needs_layout_passes=False),
)
def _sc_scan(tt_hbm, ttail_hbm, idx_hbm, cdata_hbm, cdst_hbm, u_all, my_u,
             my_k, slabs, pend_u, pend_k, mbuf, mdst, trash_v, tail_v, cnt_s,
             sem, fsem):
    cid = lax.axis_index("c")
    sid = lax.axis_index("s")
    wid = sid * _NC + cid

    @pl.when(sid == 0)
    def _():
        cnt_s[0] = 0

    tr16 = jnp.full((16,), _TRASH, jnp.int32)
    mdst[pl.ds(0, 16)] = tr16
    mdst[pl.ds(16, 16)] = tr16
    for j in range(1536 // 16):
        trash_v[pl.ds(j * 16, 16)] = tr16
    pltpu.sync_copy(trash_v, cdst_hbm.at[pl.ds(cid * _PERC + sid * 1536, 1536)])
    plsc.subcore_barrier()

    pltpu.sync_copy(idx_hbm, u_all)

    base_ns = _NSLAB // _NW
    extra = _NSLAB - base_ns * _NW
    ns = base_ns + jnp.where(wid < extra, 1, 0)
    s0 = wid * base_ns + jnp.minimum(wid, extra)
    row_lo = s0 * _SLABW
    row_hi = jnp.minimum((s0 + ns) * _SLABW, NROWS)

    def prefilter(g, m):
        uv = plsc.load_gather(u_all, [g * 16 + _iota16()])
        msk = (uv >= row_lo) & (uv < row_hi)
        rank = plsc.cumsum(msk.astype(jnp.int32)) - 1
        pos = m + rank
        plsc.store_scatter(my_u, [pos], uv, mask=msk)
        plsc.store_scatter(my_k, [pos], _iota16() + g * 16, mask=msk)
        return m + plsc.all_reduce_population_count(msk)[0]

    m = lax.fori_loop(0, BATCH // 16, prefilter, jnp.int32(0))
    ngrp = (m + 15) // 16

    def slab_start(si):
        return pl.multiple_of(
            jnp.minimum((s0 + si) * _SLABW, _ALIGNED_LAST), 128)

    def start_slab_dma(si, buf):
        pltpu.async_copy(
            tt_hbm.at[:, pl.ds(slab_start(si), _SLABW)], slabs.at[buf], sem
        )

    def wait_slab_dma(buf):
        pltpu.make_async_copy(
            tt_hbm.at[:, pl.ds(0, _SLABW)], slabs.at[buf], sem
        ).wait()

    def wait_flush():
        pltpu.make_async_copy(
            mbuf.at[pl.ds(0, 16 * D)], cdata_hbm.at[pl.ds(0, 16 * D)], fsem
        ).wait()
        pltpu.make_async_copy(
            mdst.at[pl.ds(0, 16)], cdst_hbm.at[pl.ds(0, 16)], fsem
        ).wait()

    def issue_flush(half, nflush):
        lax.cond(nflush > 0, lambda _: (wait_flush(), 0)[1], lambda _: 0, 0)
        pos = plsc.fetch_and_add(cnt_s.at[0], jnp.int32(16), subcore_id=0)
        gpos = pl.multiple_of(cid * _PERC + pos, 16)
        pltpu.async_copy(
            mbuf.at[pl.ds(half * (16 * D), 16 * D)],
            cdata_hbm.at[pl.ds(gpos * D, 16 * D)], fsem)
        pltpu.async_copy(
            mdst.at[pl.ds(half * 16, 16)], cdst_hbm.at[pl.ds(gpos, 16)], fsem)

    def extract(n, loader, state):
        mf, half, nflush = state
        lvec = pend_u[pl.ds(0, 16)]
        kvec = pend_k[pl.ds(0, 16)]
        valid = _iota16() < n
        pos16 = mf + _iota16()
        newhalf = jnp.where(pos16 >= 16, 1 - half, half)
        rows = (pos16 & 15) + newhalf * 16
        for c in range(D):
            vals = loader(c, lvec)
            plsc.store_scatter(mbuf, [rows * D + c], vals, mask=valid)
        plsc.store_scatter(mdst, [rows], kvec, mask=valid)
        mf2 = mf + n
        crossed = mf2 >= 16

        def do_flush(nf):
            issue_flush(half, nf)
            return nf + 1

        nflush = lax.cond(crossed, do_flush, lambda nf: nf, nflush)
        return (mf2 & 15, jnp.where(crossed, 1 - half, half), nflush)

    start_slab_dma(0, jnp.int32(0))

    def slab_body(si, carry):
        f, mstate = carry
        buf = si % 2

        @pl.when(si + 1 < ns)
        def _():
            start_slab_dma(si + 1, (si + 1) % 2)

        wait_slab_dma(buf)
        lo = slab_start(si)
        loader = lambda c, lv: plsc.load_gather(
            slabs, [_splat(buf), _splat(jnp.int32(c)), lv])
        return select_and_extract(lo, _SLABW, loader, (f, mstate))

    def select_and_extract(lo, width, loader, carry):
        f, mstate = carry

        def sel_body(g, carry2):
            f2, mstate2 = carry2
            uv = plsc.load_gather(my_u, [g * 16 + _iota16()])
            kv = plsc.load_gather(my_k, [g * 16 + _iota16()])
            msk = (uv >= lo) & (uv < lo + width)
            cnt = plsc.all_reduce_population_count(msk)[0]

            def append(c3):
                f3, ms3 = c3
                rank = plsc.cumsum(msk.astype(jnp.int32)) - 1
                plsc.store_scatter(pend_u, [f3 + rank], uv - lo, mask=msk)
                plsc.store_scatter(pend_k, [f3 + rank], kv, mask=msk)
                f4 = f3 + cnt

                def do_ext(c4):
                    f5, ms5 = c4
                    ms6 = extract(jnp.int32(16), loader, ms5)
                    rem = f5 - 16
                    sh_u = pend_u[pl.ds(16, 16)]
                    sh_k = pend_k[pl.ds(16, 16)]
                    shm = _iota16() < rem
                    plsc.store_scatter(pend_u, [_iota16()], sh_u, mask=shm)
                    plsc.store_scatter(pend_k, [_iota16()], sh_k, mask=shm)
                    return (rem, ms6)

                return lax.cond(f4 >= 16, do_ext, lambda c4: c4, (f4, ms3))

            return lax.cond(cnt > 0, append, lambda c3: c3, (f2, mstate2))

        f, mstate = lax.fori_loop(0, ngrp, sel_body, (f, mstate))
        f, mstate = lax.cond(
            f > 0,
            lambda c2: (jnp.int32(0), extract(c2[0], loader, c2[1])),
            lambda c2: c2, (f, mstate))
        return (f, mstate)

    _, mstate = lax.fori_loop(
        0, ns, slab_body,
        (jnp.int32(0), (jnp.int32(0), jnp.int32(0), jnp.int32(0))))

    pltpu.sync_copy(ttail_hbm, tail_v)
    tail_loader = lambda c, lv: plsc.load_gather(
        tail_v, [_splat(jnp.int32(c)), lv])
    _, mstate = select_and_extract(
        jnp.int32(_TAIL0), jnp.int32(_NTAIL), tail_loader,
        (jnp.int32(0), mstate))

    mf, half, nflush = mstate

    def final_flush(c):
        mf2, half2, nf2 = c
        plsc.store_scatter(mdst, [half2 * 16 + _iota16()], tr16,
                           mask=_iota16() >= mf2)
        issue_flush(half2, nf2)
        return (mf2, half2, nf2 + 1)

    mf, half, nflush = lax.cond(mf > 0, final_flush, lambda c: c,
                                (mf, half, nflush))
    lax.cond(nflush > 0, lambda _: (wait_flush(), 0)[1], lambda _: 0, 0)


_BPB = _CAP // _NW


@functools.partial(
    pl.kernel,
    mesh=_sc_mesh,
    out_type=jax.ShapeDtypeStruct((BATCH + 8, D), jnp.float32),
    scratch_types=[
        pltpu.VMEM((_BPB,), jnp.int32),
        pltpu.VMEM((_BPB, D), jnp.float32),
        pltpu.SemaphoreType.DMA,
    ],
    compiler_params=pltpu.CompilerParams(use_tc_tiling_on_sc=False),
)
def _sc_scatter(cdata_hbm, cdst_hbm, out_hbm, kidx_v, rows_v, sem):
    wid = lax.axis_index("s") * _NC + lax.axis_index("c")
    base = wid * _BPB
    pltpu.sync_copy(cdst_hbm.at[pl.ds(base, _BPB)], kidx_v)
    pltpu.sync_copy(cdata_hbm.at[pl.ds(base, _BPB)], rows_v)
    pltpu.async_copy(rows_v, out_hbm.at[kidx_v], sem).wait()


_BM = 512


def _mm_body(x_ref, w_ref, b_ref, o_ref):
    o_ref[...] = (
        jnp.dot(x_ref[...], w_ref[...], preferred_element_type=jnp.float32)
        + b_ref[...]
    )


def _tc_matmul(images, Wt, b):
    return pl.pallas_call(
        _mm_body,
        grid=(BATCH // _BM,),
        in_specs=[
            pl.BlockSpec((_BM, IMG_DIM), lambda i: (i, 0)),
            pl.BlockSpec((IMG_DIM, D), lambda i: (0, 0)),
            pl.BlockSpec((1, D), lambda i: (0, 0)),
        ],
        out_specs=pl.BlockSpec((_BM, D), lambda i: (i, 0)),
        out_shape=jax.ShapeDtypeStruct((BATCH, D), jnp.float32),
    )(images, Wt, b.reshape(1, D))


def kernel(users, images, emb_table, W, b):
    tt = emb_table.T
    ttail = tt[:, _TAIL0:]
    cdata, cdst = _sc_scan(tt, ttail, users.astype(jnp.int32))
    scattered = _sc_scatter(cdata.reshape(_CAP, D), cdst)
    u_emb = scattered[:BATCH]
    img_emb = _tc_matmul(images, W.T, b)
    return (u_emb, img_emb)


# speedup vs baseline: 1.3874x; 1.3874x over previous
import functools

import jax
import jax.numpy as jnp
from jax import lax
from jax.experimental import pallas as pl
from jax.experimental.pallas import tpu as pltpu
from jax.experimental.pallas import tpu_sc as plsc

D = 64
IMG_DIM = 1536
BATCH = 16384

_info = plsc.get_sparse_core_info()
_NC = _info.num_cores
_NS = _info.num_subcores
_NW = _NC * _NS
_BPW = BATCH // _NW

_sc_mesh = plsc.VectorSubcoreMesh(core_axis_name="c", subcore_axis_name="s")


@functools.partial(
    pl.kernel,
    mesh=_sc_mesh,
    out_type=jax.ShapeDtypeStruct((BATCH, D), jnp.float32),
    scratch_types=[
        pltpu.VMEM((_BPW,), jnp.int32),
        pltpu.SemaphoreType.DMA,
    ],
)
def _sc_gather(table_hbm, idx_hbm, out_hbm, idx_s, sem):
    wid = lax.axis_index("s") * _NC + lax.axis_index("c")
    base = wid * _BPW
    pltpu.sync_copy(idx_hbm.at[pl.ds(base, _BPW)], idx_s)

    def issue(g, _):
        grp = idx_s[pl.ds(g * 16, 16)]
        for j in range(16):
            u = grp[j]
            pltpu.async_copy(table_hbm.at[u], out_hbm.at[base + g * 16 + j], sem)
        return 0

    lax.fori_loop(0, _BPW // 16, issue, 0)

    def drain(k, _):
        pltpu.make_async_copy(table_hbm.at[0], out_hbm.at[base], sem).wait()
        return 0

    lax.fori_loop(0, _BPW, drain, 0)


_BM = 512


def _mm_body(x_ref, w_ref, b_ref, o_ref):
    o_ref[...] = (
        jnp.dot(x_ref[...], w_ref[...], preferred_element_type=jnp.float32)
        + b_ref[...]
    )


def _tc_matmul(images, Wt, b):
    return pl.pallas_call(
        _mm_body,
        grid=(BATCH // _BM,),
        in_specs=[
            pl.BlockSpec((_BM, IMG_DIM), lambda i: (i, 0)),
            pl.BlockSpec((IMG_DIM, D), lambda i: (0, 0)),
            pl.BlockSpec((1, D), lambda i: (0, 0)),
        ],
        out_specs=pl.BlockSpec((_BM, D), lambda i: (i, 0)),
        out_shape=jax.ShapeDtypeStruct((BATCH, D), jnp.float32),
    )(images, Wt, b.reshape(1, D))


def kernel(users, images, emb_table, W, b):
    u_emb = _sc_gather(emb_table, users.astype(jnp.int32))
    img_emb = _tc_matmul(images, W.T, b)
    return (u_emb, img_emb)
